# Initial kernel scaffold; baseline (speedup 1.0000x reference)
import functools
import jax
import jax.numpy as jnp
from jax import lax
from jax.experimental import pallas as pl
from jax.experimental.pallas import tpu as pltpu
from jax.experimental.pallas import tpu_sc as plsc

NC = 2
NS = 16
NW = NC * NS
LANES = 16

B = 16384
L = 50
D = 64

S_CHUNK = 8
R_CHUNK = S_CHUNK * L
G = 100
N_DMA = R_CHUNK // G
SEQ_PER_W = B // NW
CHUNKS = SEQ_PER_W // S_CHUNK


def _embed_body(x_hbm, xlen_hbm, w_hbm, raw_hbm, ret_hbm,
                idx_v, rows_v, acc_v, xlen_v, sem):
    wid = lax.axis_index("s") * NC + lax.axis_index("c")
    seq_base = wid * SEQ_PER_W

    pltpu.sync_copy(xlen_hbm.at[pl.ds(seq_base, SEQ_PER_W)], xlen_v)

    def chunk_body(c, carry):
        seq0 = seq_base + c * S_CHUNK
        row0 = seq0 * L
        pltpu.sync_copy(x_hbm.at[pl.ds(seq0 * L // G, N_DMA)], idx_v)

        copies = []
        for j in range(N_DMA):
            cp = pltpu.make_async_copy(
                w_hbm.at[idx_v.at[j]],
                rows_v.at[pl.ds(j * G, G)],
                sem)
            cp.start()
            copies.append(cp)
        for cp in copies:
            cp.wait()

        pltpu.sync_copy(rows_v, raw_hbm.at[pl.ds(row0, R_CHUNK)])

        for s in range(S_CHUNK):
            def red_body(r, acc):
                base = s * L + r
                return tuple(
                    acc[k] + rows_v[base, pl.ds(k * LANES, LANES)]
                    for k in range(D // LANES))
            zero = jnp.zeros((LANES,), jnp.float32)
            acc = lax.fori_loop(0, L, red_body, (zero,) * (D // LANES))
            lane_idx = jnp.full((LANES,), c * S_CHUNK + s, jnp.int32)
            xl = plsc.load_gather(xlen_v, [lane_idx])
            for k in range(D // LANES):
                acc_v[s, pl.ds(k * LANES, LANES)] = acc[k] / xl

        pltpu.sync_copy(acc_v, ret_hbm.at[pl.ds(seq0, S_CHUNK)])
        return carry

    lax.fori_loop(0, CHUNKS, chunk_body, 0)


_embed_kernel = functools.partial(
    pl.kernel,
    out_type=(jax.ShapeDtypeStruct((B * L, D), jnp.float32),
              jax.ShapeDtypeStruct((B, D), jnp.float32)),
    mesh=plsc.VectorSubcoreMesh(core_axis_name="c", subcore_axis_name="s"),
    scratch_types=[
        pltpu.VMEM((N_DMA, G), jnp.int32),
        pltpu.VMEM((R_CHUNK, D), jnp.float32),
        pltpu.VMEM((S_CHUNK, D), jnp.float32),
        pltpu.VMEM((SEQ_PER_W,), jnp.float32),
        pltpu.SemaphoreType.DMA,
    ],
)(_embed_body)


def kernel(x, x_len, weight):
    x2d = x.reshape(B * L // G, G).astype(jnp.int32)
    xlen = x_len.reshape(B).astype(jnp.float32)
    raw_flat, ret = _embed_kernel(x2d, xlen, weight)
    return (ret, raw_flat.reshape(B, L, D))


# speedup vs baseline: 1.8508x; 1.8508x over previous
import functools
import jax
import jax.numpy as jnp
from jax import lax
from jax.experimental import pallas as pl
from jax.experimental.pallas import tpu as pltpu
from jax.experimental.pallas import tpu_sc as plsc

NC = 2
NS = 16
NW = NC * NS
LANES = 16

B = 16384
L = 50
D = 64

S_CHUNK = 16
R_CHUNK = S_CHUNK * L
G = 100
N_DMA = R_CHUNK // G
SEQ_PER_W = B // NW
CHUNKS = SEQ_PER_W // S_CHUNK


def _broadcast_lane(vec, lane):
    idx = jnp.full((LANES, 1), lane, jnp.int32)
    dnums = lax.GatherDimensionNumbers(
        offset_dims=(), collapsed_slice_dims=(0,), start_index_map=(0,))
    return lax.gather(vec, idx, dnums, (1,),
                      mode=lax.GatherScatterMode.PROMISE_IN_BOUNDS)


def _embed_body(x_hbm, xlen_hbm, w_hbm, raw_hbm, ret_hbm,
                idx_v, rows_v, acc_v, xlen_v, sem):
    wid = lax.axis_index("s") * NC + lax.axis_index("c")
    seq_base = wid * SEQ_PER_W

    pltpu.sync_copy(xlen_hbm.at[pl.ds(seq_base, SEQ_PER_W)], xlen_v)

    def chunk_body(c, carry):
        seq0 = pl.multiple_of(seq_base + c * S_CHUNK, S_CHUNK)
        row0 = pl.multiple_of(seq0 * L, R_CHUNK)
        pltpu.sync_copy(x_hbm.at[pl.ds(pl.multiple_of(seq0 * L // G, N_DMA), N_DMA)],
                        idx_v)

        copies = []
        for j in range(N_DMA):
            cp = pltpu.make_async_copy(
                w_hbm.at[idx_v.at[j]],
                rows_v.at[pl.ds(j * G, G)],
                sem)
            cp.start()
            copies.append(cp)
        for cp in copies:
            cp.wait()

        pltpu.sync_copy(rows_v, raw_hbm.at[pl.ds(row0, R_CHUNK)])

        inv = 1.0 / xlen_v[pl.ds(pl.multiple_of(c * S_CHUNK, S_CHUNK), LANES)]
        for s in range(S_CHUNK):
            def red_body(r, acc):
                base = s * L + r
                return tuple(
                    acc[k] + rows_v[base, pl.ds(k * LANES, LANES)]
                    for k in range(D // LANES))
            zero = jnp.zeros((LANES,), jnp.float32)
            acc = lax.fori_loop(0, L, red_body, (zero,) * (D // LANES))
            xl = _broadcast_lane(inv, s)
            for k in range(D // LANES):
                acc_v[s, pl.ds(k * LANES, LANES)] = acc[k] * xl

        pltpu.sync_copy(acc_v, ret_hbm.at[pl.ds(seq0, S_CHUNK)])
        return carry

    lax.fori_loop(0, CHUNKS, chunk_body, 0)


_embed_kernel = functools.partial(
    pl.kernel,
    out_type=(jax.ShapeDtypeStruct((B * L, D), jnp.float32),
              jax.ShapeDtypeStruct((B, D), jnp.float32)),
    mesh=plsc.VectorSubcoreMesh(core_axis_name="c", subcore_axis_name="s"),
    compiler_params=pltpu.CompilerParams(use_tc_tiling_on_sc=False),
    scratch_types=[
        pltpu.VMEM((N_DMA, G), jnp.int32),
        pltpu.VMEM((R_CHUNK, D), jnp.float32),
        pltpu.VMEM((S_CHUNK, D), jnp.float32),
        pltpu.VMEM((SEQ_PER_W,), jnp.float32),
        pltpu.SemaphoreType.DMA,
    ],
)(_embed_body)


def kernel(x, x_len, weight):
    x2d = x.reshape(B * L // G, G).astype(jnp.int32)
    xlen = x_len.reshape(B).astype(jnp.float32)
    raw_flat, ret = _embed_kernel(x2d, xlen, weight)
    return (ret, raw_flat.reshape(B, L, D))


# speedup vs baseline: 2.0064x; 1.0841x over previous
import functools
import jax
import jax.numpy as jnp
from jax import lax
from jax.experimental import pallas as pl
from jax.experimental.pallas import tpu as pltpu
from jax.experimental.pallas import tpu_sc as plsc

NC = 2
NS = 16
NW = NC * NS
LANES = 16

B = 16384
L = 50
D = 64

S_CHUNK = 16
R_CHUNK = S_CHUNK * L
G = 100
N_DMA = R_CHUNK // G
SEQ_PER_W = B // NW
CHUNKS = SEQ_PER_W // S_CHUNK
KCOL = D // LANES


def _broadcast_lane(vec, lane):
    idx = jnp.full((LANES, 1), lane, jnp.int32)
    dnums = lax.GatherDimensionNumbers(
        offset_dims=(), collapsed_slice_dims=(0,), start_index_map=(0,))
    return lax.gather(vec, idx, dnums, (1,),
                      mode=lax.GatherScatterMode.PROMISE_IN_BOUNDS)


def _embed_body(x_hbm, xlen_hbm, w_hbm, raw_hbm, ret_hbm,
                idx0, idx1, rows0, rows1, acc_v, xlen_v,
                sem_g0, sem_g1, sem_o0, sem_o1):
    idx = (idx0, idx1)
    rows = (rows0, rows1)
    sem_g = (sem_g0, sem_g1)
    sem_o = (sem_o0, sem_o1)

    wid = lax.axis_index("s") * NC + lax.axis_index("c")
    seq_base = wid * SEQ_PER_W

    pltpu.sync_copy(xlen_hbm.at[pl.ds(seq_base, SEQ_PER_W)], xlen_v)

    def fire_chunk(c, b):
        x_row0 = pl.multiple_of((seq_base + c * S_CHUNK) * L // G, N_DMA)
        pltpu.sync_copy(x_hbm.at[pl.ds(x_row0, N_DMA)], idx[b])
        for j in range(N_DMA):
            pltpu.make_async_copy(
                w_hbm.at[idx[b].at[j]],
                rows[b].at[pl.ds(j * G, G)],
                sem_g[b]).start()

    def drain_gather(b):
        pltpu.make_async_copy(
            raw_hbm.at[pl.ds(0, R_CHUNK)], rows[b], sem_g[b]).wait()

    def raw_out_copy(c, b):
        row0 = pl.multiple_of((seq_base + c * S_CHUNK) * L, R_CHUNK)
        return pltpu.make_async_copy(
            rows[b], raw_hbm.at[pl.ds(row0, R_CHUNK)], sem_o[b])

    def drain_out(b):
        raw_out_copy(0, b).wait()

    fire_chunk(0, 0)

    def outer_body(o, carry):
        for b in range(2):
            c = o * 2 + b
            b2 = 1 - b
            drain_gather(b)
            raw_out_copy(c, b).start()

            @pl.when(c + 1 < CHUNKS)
            def _prefetch():
                @pl.when(c >= 1)
                def _():
                    drain_out(b2)
                fire_chunk(c + 1, b2)

            seq_lo = pl.multiple_of(c * S_CHUNK, S_CHUNK)
            inv = 1.0 / xlen_v[pl.ds(seq_lo, LANES)]
            for s in range(S_CHUNK):
                def red_body(r, acc):
                    base = s * L + r
                    return tuple(
                        acc[k] + rows[b][base, pl.ds(k * LANES, LANES)]
                        for k in range(KCOL))
                zero = jnp.zeros((LANES,), jnp.float32)
                acc = lax.fori_loop(0, L, red_body, (zero,) * KCOL,
                                    unroll=2)
                xl = _broadcast_lane(inv, s)
                for k in range(KCOL):
                    acc_v[s, pl.ds(k * LANES, LANES)] = acc[k] * xl
            seq0 = pl.multiple_of(seq_base + c * S_CHUNK, S_CHUNK)
            pltpu.sync_copy(acc_v, ret_hbm.at[pl.ds(seq0, S_CHUNK)])
        return carry

    lax.fori_loop(0, CHUNKS // 2, outer_body, 0)
    drain_out(0)
    drain_out(1)


_embed_kernel = functools.partial(
    pl.kernel,
    out_type=(jax.ShapeDtypeStruct((B * L, D), jnp.float32),
              jax.ShapeDtypeStruct((B, D), jnp.float32)),
    mesh=plsc.VectorSubcoreMesh(core_axis_name="c", subcore_axis_name="s"),
    compiler_params=pltpu.CompilerParams(use_tc_tiling_on_sc=False),
    scratch_types=[
        pltpu.VMEM((N_DMA, G), jnp.int32),
        pltpu.VMEM((N_DMA, G), jnp.int32),
        pltpu.VMEM((R_CHUNK, D), jnp.float32),
        pltpu.VMEM((R_CHUNK, D), jnp.float32),
        pltpu.VMEM((S_CHUNK, D), jnp.float32),
        pltpu.VMEM((SEQ_PER_W,), jnp.float32),
        pltpu.SemaphoreType.DMA,
        pltpu.SemaphoreType.DMA,
        pltpu.SemaphoreType.DMA,
        pltpu.SemaphoreType.DMA,
    ],
)(_embed_body)


def kernel(x, x_len, weight):
    x2d = x.reshape(B * L // G, G).astype(jnp.int32)
    xlen = x_len.reshape(B).astype(jnp.float32)
    raw_flat, ret = _embed_kernel(x2d, xlen, weight)
    return (ret, raw_flat.reshape(B, L, D))
